# Initial kernel scaffold; baseline (speedup 1.0000x reference)
import jax
import jax.numpy as jnp
from jax.experimental import pallas as pl


def kernel(x, tfidf, ids, W1, b1, W2, b2, Wmu, bmu, Wlv, blv, beta, mean_prior, var_prior, log_var_prior):
    raise NotImplementedError("write your pallas kernel here")


# speedup vs baseline: 9.4741x; 9.4741x over previous
import jax
import jax.numpy as jnp
from jax import lax
from jax.experimental import pallas as pl

B = 1024
V = 100000
H = 512
T = 50
KSEL = 512

VC = 2048
NCHUNK = (V + VC - 1) // VC
VC5 = 1024
NCHUNK5 = (V + VC5 - 1) // VC5
RB = 16


def _softplus(a):
    return jnp.maximum(a, 0.0) + jnp.log1p(jnp.exp(-jnp.abs(a)))


def _bn(a):
    m = jnp.mean(a, axis=0, keepdims=True)
    v = jnp.mean((a - m) ** 2, axis=0, keepdims=True)
    return (a - m) / jnp.sqrt(v + 1e-5)


def _k1_body(x_ref, w1_ref, b1_ref, out_ref):
    i = pl.program_id(0)
    jcol = i * VC + lax.broadcasted_iota(jnp.int32, (1, VC), 1)
    valid = jcol < V
    validr = (i * VC + lax.broadcasted_iota(jnp.int32, (VC, 1), 0)) < V
    xm = jnp.where(valid, x_ref[...], 0.0)
    w1m = jnp.where(validr, w1_ref[...], 0.0)

    @pl.when(i == 0)
    def _():
        out_ref[...] = jnp.broadcast_to(b1_ref[...], (B, H))

    out_ref[...] += jnp.dot(xm, w1m, preferred_element_type=jnp.float32)


def _matmul_xw1(x, W1, b1):
    return pl.pallas_call(
        _k1_body,
        grid=(NCHUNK,),
        in_specs=[
            pl.BlockSpec((B, VC), lambda i: (0, i)),
            pl.BlockSpec((VC, H), lambda i: (i, 0)),
            pl.BlockSpec((1, H), lambda i: (0, 0)),
        ],
        out_specs=pl.BlockSpec((B, H), lambda i: (0, 0)),
        out_shape=jax.ShapeDtypeStruct((B, H), jnp.float32),
    )(x, W1, b1.reshape(1, H))


def _k2_body(a1_ref, w2_ref, b2_ref, wmu_ref, bmu_ref, wlv_ref, blv_ref,
             eps_ref, mu_ref, lv_ref, var_ref, z_ref, theta_ref):
    h1 = _softplus(a1_ref[...])
    h2 = _softplus(jnp.dot(h1, w2_ref[...],
                           preferred_element_type=jnp.float32) + b2_ref[...])
    mu = _bn(jnp.dot(h2, wmu_ref[...],
                     preferred_element_type=jnp.float32) + bmu_ref[...])
    lv = _bn(jnp.dot(h2, wlv_ref[...],
                     preferred_element_type=jnp.float32) + blv_ref[...])
    z = mu + eps_ref[...] * jnp.exp(0.5 * lv)
    zs = z - jnp.max(z, axis=1, keepdims=True)
    ez = jnp.exp(zs)
    theta = ez / jnp.sum(ez, axis=1, keepdims=True)
    mu_ref[...] = mu
    lv_ref[...] = lv
    var_ref[...] = jnp.exp(lv)
    z_ref[...] = z
    theta_ref[...] = theta


def _enc_tail(a1, W2, b2, Wmu, bmu, Wlv, blv, eps):
    shapes = [jax.ShapeDtypeStruct((B, T), jnp.float32)] * 3 + \
             [jax.ShapeDtypeStruct((B, T), jnp.float32)] * 2
    return pl.pallas_call(
        _k2_body,
        out_shape=(jax.ShapeDtypeStruct((B, T), jnp.float32),
                   jax.ShapeDtypeStruct((B, T), jnp.float32),
                   jax.ShapeDtypeStruct((B, T), jnp.float32),
                   jax.ShapeDtypeStruct((B, T), jnp.float32),
                   jax.ShapeDtypeStruct((B, T), jnp.float32)),
    )(a1, W2, b2.reshape(1, H), Wmu, bmu.reshape(1, T),
      Wlv, blv.reshape(1, T), eps)


def _count_ge(v, tbits):
    t = lax.bitcast_convert_type(tbits, jnp.float32)
    return jnp.sum((v >= t).astype(jnp.float32), axis=1, keepdims=True)


def _count_le(v, tbits):
    t = lax.bitcast_convert_type(tbits, jnp.float32)
    return jnp.sum((v <= t).astype(jnp.float32), axis=1, keepdims=True)


def _k3_body(tf_ref, thi_ref, jhi_ref, tlo_ref, jlo_ref):
    v = tf_ref[...]
    kf = jnp.float32(KSEL)
    n = jnp.float32(V)
    jcol = lax.broadcasted_iota(jnp.int32, (RB, V), 1)

    lo = jnp.zeros((RB, 1), jnp.int32)
    hi = jnp.full((RB, 1), 0x3F800000, jnp.int32)
    c_lo = jnp.full((RB, 1), n)
    c_hi = jnp.zeros((RB, 1), jnp.float32)

    def top_it(_, carry):
        lo, hi, c_lo, c_hi = carry
        mid = (lo + hi) >> 1
        c = _count_ge(v, mid)
        ge = c >= kf
        return (jnp.where(ge, mid, lo), jnp.where(ge, hi, mid),
                jnp.where(ge, c, c_lo), jnp.where(ge, c_hi, c))

    lo, hi, c_lo, c_hi = lax.fori_loop(0, 30, top_it, (lo, hi, c_lo, c_hi))
    t_hi = lax.bitcast_convert_type(lo, jnp.float32)
    d = kf - c_hi
    eq_hi = v == t_hi

    jl = jnp.zeros((RB, 1), jnp.int32)
    jh = jnp.full((RB, 1), 1 << 17, jnp.int32)

    def topj_it(_, carry):
        jl, jh = carry
        mid = (jl + jh) >> 1
        g = jnp.sum((eq_hi & (jcol >= mid)).astype(jnp.float32),
                    axis=1, keepdims=True)
        ok = g >= d
        return jnp.where(ok, mid, jl), jnp.where(ok, jh, mid)

    jl, _ = lax.fori_loop(0, 17, topj_it, (jl, jh))

    lo2 = jnp.full((RB, 1), -1, jnp.int32)
    hi2 = jnp.full((RB, 1), 0x3F800000, jnp.int32)
    c2_lo = jnp.zeros((RB, 1), jnp.float32)
    c2_hi = jnp.full((RB, 1), n)

    def bot_it(_, carry):
        lo2, hi2, c2_lo, c2_hi = carry
        mid = (lo2 + hi2) >> 1
        c = _count_le(v, mid)
        ge = c >= kf
        return (jnp.where(ge, lo2, mid), jnp.where(ge, mid, hi2),
                jnp.where(ge, c2_lo, c), jnp.where(ge, c, c2_hi))

    lo2, hi2, c2_lo, c2_hi = lax.fori_loop(0, 31, bot_it,
                                           (lo2, hi2, c2_lo, c2_hi))
    t_lo = lax.bitcast_convert_type(hi2, jnp.float32)
    d2 = kf - c2_lo
    eq_lo = v == t_lo

    jl2 = jnp.full((RB, 1), -1, jnp.int32)
    jh2 = jnp.full((RB, 1), (1 << 17) - 1, jnp.int32)

    def botj_it(_, carry):
        jl2, jh2 = carry
        mid = (jl2 + jh2) >> 1
        g = jnp.sum((eq_lo & (jcol <= mid)).astype(jnp.float32),
                    axis=1, keepdims=True)
        ok = g >= d2
        return jnp.where(ok, jl2, mid), jnp.where(ok, mid, jh2)

    _, jh2 = lax.fori_loop(0, 17, botj_it, (jl2, jh2))

    thi_ref[...] = jnp.broadcast_to(t_hi, (RB, 128))
    jhi_ref[...] = jnp.broadcast_to(jl, (RB, 128))
    tlo_ref[...] = jnp.broadcast_to(t_lo, (RB, 128))
    jlo_ref[...] = jnp.broadcast_to(jh2, (RB, 128))


def _selection(tfidf):
    return pl.pallas_call(
        _k3_body,
        grid=(B // RB,),
        in_specs=[pl.BlockSpec((RB, V), lambda i: (i, 0))],
        out_specs=(pl.BlockSpec((RB, 128), lambda i: (i, 0)),
                   pl.BlockSpec((RB, 128), lambda i: (i, 0)),
                   pl.BlockSpec((RB, 128), lambda i: (i, 0)),
                   pl.BlockSpec((RB, 128), lambda i: (i, 0))),
        out_shape=(jax.ShapeDtypeStruct((B, 128), jnp.float32),
                   jax.ShapeDtypeStruct((B, 128), jnp.int32),
                   jax.ShapeDtypeStruct((B, 128), jnp.float32),
                   jax.ShapeDtypeStruct((B, 128), jnp.int32)),
    )(tfidf)


def _k4_body(theta_ref, beta_ref, m_ref, s_ref):
    i = pl.program_id(0)
    jcol = i * VC + lax.broadcasted_iota(jnp.int32, (1, VC), 1)
    valid = jcol < V
    logits = jnp.dot(theta_ref[...], beta_ref[...],
                     preferred_element_type=jnp.float32)
    cm = jnp.mean(logits, axis=0, keepdims=True)
    cv = jnp.mean((logits - cm) ** 2, axis=0, keepdims=True)
    bnl = (logits - cm) / jnp.sqrt(cv + 1e-5)
    bnl = jnp.where(valid, bnl, -jnp.inf)

    @pl.when(i == 0)
    def _():
        m_ref[...] = jnp.full((B, 128), -jnp.inf, jnp.float32)
        s_ref[...] = jnp.zeros((B, 128), jnp.float32)

    m_old = m_ref[:, 0:1]
    s_old = s_ref[:, 0:1]
    m_new = jnp.maximum(m_old, jnp.max(bnl, axis=1, keepdims=True))
    s_new = s_old * jnp.exp(m_old - m_new) + jnp.sum(
        jnp.where(valid, jnp.exp(bnl - m_new), 0.0), axis=1, keepdims=True)
    m_ref[...] = jnp.broadcast_to(m_new, (B, 128))
    s_ref[...] = jnp.broadcast_to(s_new, (B, 128))


def _decode_stats(theta, beta):
    return pl.pallas_call(
        _k4_body,
        grid=(NCHUNK,),
        in_specs=[pl.BlockSpec((B, T), lambda i: (0, 0)),
                  pl.BlockSpec((T, VC), lambda i: (0, i))],
        out_specs=(pl.BlockSpec((B, 128), lambda i: (0, 0)),
                   pl.BlockSpec((B, 128), lambda i: (0, 0))),
        out_shape=(jax.ShapeDtypeStruct((B, 128), jnp.float32),
                   jax.ShapeDtypeStruct((B, 128), jnp.float32)),
    )(theta, beta)


def _k5_body(theta_ref, beta_ref, m_ref, s_ref, x_ref, tf_ref, w1_ref,
             thi_ref, jhi_ref, tlo_ref, jlo_ref,
             xr_ref, accn_ref, accp_ref):
    i = pl.program_id(0)
    jcol = i * VC5 + lax.broadcasted_iota(jnp.int32, (1, VC5), 1)
    valid = jcol < V
    logits = jnp.dot(theta_ref[...], beta_ref[...],
                     preferred_element_type=jnp.float32)
    cm = jnp.mean(logits, axis=0, keepdims=True)
    cv = jnp.mean((logits - cm) ** 2, axis=0, keepdims=True)
    bnl = (logits - cm) / jnp.sqrt(cv + 1e-5)
    xr = jnp.exp(bnl - m_ref[:, 0:1]) / s_ref[:, 0:1]
    xr_ref[...] = xr

    xv = x_ref[...]
    tfv = tf_ref[...]
    t_hi = thi_ref[:, 0:1]
    j_hi = jhi_ref[:, 0:1]
    t_lo = tlo_ref[:, 0:1]
    j_lo = jlo_ref[:, 0:1]
    sel_hi = ((tfv > t_hi) | ((tfv == t_hi) & (jcol >= j_hi))) & valid
    sel_lo = ((tfv < t_lo) | ((tfv == t_lo) & (jcol <= j_lo))) & valid
    validr = (i * VC5 + lax.broadcasted_iota(jnp.int32, (VC5, 1), 0)) < V
    w1m = jnp.where(validr, w1_ref[...], 0.0)
    dhi = jnp.where(sel_hi, xr - xv, 0.0)
    dlo = jnp.where(sel_lo, xr - xv, 0.0)

    @pl.when(i == 0)
    def _():
        accn_ref[...] = jnp.zeros((B, H), jnp.float32)
        accp_ref[...] = jnp.zeros((B, H), jnp.float32)

    accn_ref[...] += jnp.dot(dhi, w1m, preferred_element_type=jnp.float32)
    accp_ref[...] += jnp.dot(dlo, w1m, preferred_element_type=jnp.float32)


def _decode_and_corr(theta, beta, m, s, x, tfidf, W1, thi, jhi, tlo, jlo):
    return pl.pallas_call(
        _k5_body,
        grid=(NCHUNK5,),
        in_specs=[pl.BlockSpec((B, T), lambda i: (0, 0)),
                  pl.BlockSpec((T, VC5), lambda i: (0, i)),
                  pl.BlockSpec((B, 128), lambda i: (0, 0)),
                  pl.BlockSpec((B, 128), lambda i: (0, 0)),
                  pl.BlockSpec((B, VC5), lambda i: (0, i)),
                  pl.BlockSpec((B, VC5), lambda i: (0, i)),
                  pl.BlockSpec((VC5, H), lambda i: (i, 0)),
                  pl.BlockSpec((B, 128), lambda i: (0, 0)),
                  pl.BlockSpec((B, 128), lambda i: (0, 0)),
                  pl.BlockSpec((B, 128), lambda i: (0, 0)),
                  pl.BlockSpec((B, 128), lambda i: (0, 0))],
        out_specs=(pl.BlockSpec((B, VC5), lambda i: (0, i)),
                   pl.BlockSpec((B, H), lambda i: (0, 0)),
                   pl.BlockSpec((B, H), lambda i: (0, 0))),
        out_shape=(jax.ShapeDtypeStruct((B, V), jnp.float32),
                   jax.ShapeDtypeStruct((B, H), jnp.float32),
                   jax.ShapeDtypeStruct((B, H), jnp.float32)),
    )(theta, beta, m, s, x, tfidf, W1, thi, jhi, tlo, jlo)


def _k6_body(a1_ref, accn_ref, accp_ref, w2_ref, b2_ref, wmu_ref, bmu_ref,
             wlv_ref, blv_ref, eps1_ref, eps2_ref, zn_ref, zp_ref):
    def tail(a, eps):
        h1 = _softplus(a)
        h2 = _softplus(jnp.dot(h1, w2_ref[...],
                               preferred_element_type=jnp.float32)
                       + b2_ref[...])
        mu = _bn(jnp.dot(h2, wmu_ref[...],
                         preferred_element_type=jnp.float32) + bmu_ref[...])
        lv = _bn(jnp.dot(h2, wlv_ref[...],
                         preferred_element_type=jnp.float32) + blv_ref[...])
        return mu + eps * jnp.exp(0.5 * lv)

    a1 = a1_ref[...]
    zn_ref[...] = tail(a1 + accn_ref[...], eps1_ref[...])
    zp_ref[...] = tail(a1 + accp_ref[...], eps2_ref[...])


def _negpos_tails(a1, accn, accp, W2, b2, Wmu, bmu, Wlv, blv, eps1, eps2):
    return pl.pallas_call(
        _k6_body,
        out_shape=(jax.ShapeDtypeStruct((B, T), jnp.float32),
                   jax.ShapeDtypeStruct((B, T), jnp.float32)),
    )(a1, accn, accp, W2, b2.reshape(1, H), Wmu, bmu.reshape(1, T),
      Wlv, blv.reshape(1, T), eps1, eps2)


def kernel(x, tfidf, ids, W1, b1, W2, b2, Wmu, bmu, Wlv, blv, beta,
           mean_prior, var_prior, log_var_prior):
    kz = jax.random.key(1)
    eps0 = jax.random.normal(jax.random.fold_in(kz, 0), (B, T), jnp.float32)
    eps1 = jax.random.normal(jax.random.fold_in(kz, 1), (B, T), jnp.float32)
    eps2 = jax.random.normal(jax.random.fold_in(kz, 2), (B, T), jnp.float32)

    a1 = _matmul_xw1(x, W1, b1)
    mu, lv, var_pos, z, theta = _enc_tail(a1, W2, b2, Wmu, bmu, Wlv, blv, eps0)
    thi, jhi, tlo, jlo = _selection(tfidf)
    m, s = _decode_stats(theta, beta)
    x_recon, accn, accp = _decode_and_corr(theta, beta, m, s, x, tfidf, W1,
                                           thi, jhi, tlo, jlo)
    z_neg, z_pos = _negpos_tails(a1, accn, accp, W2, b2, Wmu, bmu, Wlv, blv,
                                 eps1, eps2)
    return (mean_prior, var_prior, log_var_prior, mu, var_pos, lv,
            x_recon, z, z_neg, z_pos)
